# baseline (speedup 1.0000x reference)
import functools

import jax
import jax.numpy as jnp
from jax import lax
from jax.experimental import pallas as pl
from jax.experimental.pallas import tpu as pltpu
from jax.experimental.pallas import tpu_sc as plsc

E = 8
TOP_K = 2
N_GROUP = 4
TOPK_GROUP = 2
H = 2048
DFF = 1024
SCALE = 2.5
T = 2048
GSZ = E // N_GROUP

BT = 256
NB = 24
NSLOT = NB * BT
NBPAD = 64

NW = 32
SLOTS_PER = NSLOT // NW
TOK_PER = T // NW
F32 = jnp.float32
I32 = jnp.int32
BF16 = jnp.bfloat16


def _lane_cumsum(v):
    n = v.shape[1]
    sh = 1
    while sh < n:
        v = v + jnp.concatenate(
            [jnp.zeros((1, sh), v.dtype), v[:, : n - sh]], axis=1)
        sh *= 2
    return v


def _router_body(x_ref, gw_ref, gb_ref, pos_ref, w_ref, bexp_ref, x16_ref):
    x = x_ref[...]
    gw = gw_ref[...]
    logits = lax.dot_general(gw, x, (((1,), (1,)), ((), ())),
                             preferred_element_type=F32)
    scores = jax.nn.sigmoid(logits) + gb_ref[...]

    s = [scores[e:e + 1, :] for e in range(E)]
    g = [jnp.maximum(s[2 * k], s[2 * k + 1]) for k in range(N_GROUP)]

    gmask = []
    for a in range(N_GROUP):
        rank = jnp.zeros_like(g[a])
        for b in range(N_GROUP):
            if b == a:
                continue
            beats = (g[b] > g[a]) if b > a else (g[b] >= g[a])
            rank = rank + beats.astype(F32)
        gmask.append((rank < float(TOPK_GROUP)).astype(F32))

    ms = [s[e] * gmask[e // GSZ] for e in range(E)]

    erank = []
    for a in range(E):
        rank = jnp.zeros_like(ms[a])
        for b in range(E):
            if b == a:
                continue
            beats = (ms[b] > ms[a]) if b > a else (ms[b] >= ms[a])
            rank = rank + beats.astype(F32)
        erank.append(rank)

    m0 = [(erank[e] == 0.0).astype(F32) for e in range(E)]
    m1 = [(erank[e] == 1.0).astype(F32) for e in range(E)]
    w0 = sum(ms[e] * m0[e] for e in range(E))
    w1 = sum(ms[e] * m1[e] for e in range(E))
    denom = w0 + w1 + 1e-6
    w0n = w0 / denom * SCALE
    w1n = w1 / denom * SCALE

    c0ex, c1ex, tot0, cnt = [], [], [], []
    for e in range(E):
        i0 = _lane_cumsum(m0[e])
        i1 = _lane_cumsum(m1[e])
        c0ex.append(i0 - m0[e])
        c1ex.append(i1 - m1[e])
        tot0.append(i0[:, T - 1:T])
        cnt.append(i0[:, T - 1:T] + i1[:, T - 1:T])

    nblk, bstart = [], []
    run = jnp.zeros((1, 1), F32)
    for e in range(E):
        bstart.append(run)
        nblk.append(jnp.floor((cnt[e] + float(BT - 1)) / float(BT)))
        run = run + nblk[e]

    pos0 = sum(m0[e] * (bstart[e] * float(BT) + c0ex[e]) for e in range(E))
    pos1 = sum(m1[e] * (bstart[e] * float(BT) + tot0[e] + c1ex[e])
               for e in range(E))

    pos_ref[...] = jnp.concatenate([pos0, pos1], axis=0).astype(I32)
    w_ref[...] = jnp.concatenate([w0n, w1n], axis=0)

    biota = lax.broadcasted_iota(I32, (1, NBPAD), 1).astype(F32)
    bexp = jnp.zeros((1, NBPAD), F32)
    for e in range(E):
        inrange = (biota >= bstart[e]) & (biota < bstart[e] + nblk[e])
        bexp = bexp + float(e) * inrange.astype(F32)
    bexp = bexp + float(E - 1) * (biota >= run).astype(F32)
    bexp = jnp.where(biota == float(NBPAD - 1), run, bexp)
    bexp_ref[...] = bexp.astype(I32)
    x16_ref[...] = x.astype(BF16)


_router = pl.pallas_call(
    _router_body,
    out_shape=(
        jax.ShapeDtypeStruct((2, T), I32),
        jax.ShapeDtypeStruct((2, T), F32),
        jax.ShapeDtypeStruct((1, NBPAD), I32),
        jax.ShapeDtypeStruct((T, H), jnp.bfloat16),
    ),
)


def _ffn_body(bexp_ref, xg_ref, gp_ref, up_ref, dp_ref, ws_ref, yg_ref):
    b = pl.program_id(0)

    @pl.when(b < bexp_ref[NBPAD - 1])
    def _():
        xb = xg_ref[...]
        gcon = (((1,), (1,)), ((), ()))
        g = lax.dot_general(xb, gp_ref[0].astype(BF16), gcon,
                            preferred_element_type=F32)
        u = lax.dot_general(xb, up_ref[0].astype(BF16), gcon,
                            preferred_element_type=F32)
        h = (g * jax.nn.sigmoid(g) * u).astype(BF16)
        y = lax.dot_general(h, dp_ref[0].astype(BF16), gcon,
                            preferred_element_type=F32)
        yg_ref[...] = y * ws_ref[0]


_ffn = pl.pallas_call(
    _ffn_body,
    grid_spec=pltpu.PrefetchScalarGridSpec(
        num_scalar_prefetch=1,
        grid=(NB,),
        in_specs=[
            pl.BlockSpec((BT, H), lambda b, bexp: (b, 0)),
            pl.BlockSpec((1, DFF, H), lambda b, bexp: (bexp[b], 0, 0)),
            pl.BlockSpec((1, DFF, H), lambda b, bexp: (bexp[b], 0, 0)),
            pl.BlockSpec((1, H, DFF), lambda b, bexp: (bexp[b], 0, 0)),
            pl.BlockSpec((1, BT, 1), lambda b, bexp: (b, 0, 0)),
        ],
        out_specs=pl.BlockSpec((BT, H), lambda b, bexp: (b, 0)),
    ),
    out_shape=jax.ShapeDtypeStruct((NSLOT, H), F32),
)


def _shared_body(x_ref, sg_ref, su_ref, sd_ref, out_ref):
    xb = x_ref[...].astype(BF16)
    gcon = (((1,), (1,)), ((), ()))
    g = lax.dot_general(xb, sg_ref[...].astype(BF16), gcon,
                        preferred_element_type=F32)
    u = lax.dot_general(xb, su_ref[...].astype(BF16), gcon,
                        preferred_element_type=F32)
    h = (g * jax.nn.sigmoid(g) * u).astype(BF16)
    out_ref[...] = lax.dot_general(h, sd_ref[...].astype(BF16), gcon,
                                   preferred_element_type=F32)


_shared = pl.pallas_call(
    _shared_body,
    grid=(T // BT,),
    in_specs=[
        pl.BlockSpec((BT, H), lambda b: (b, 0)),
        pl.BlockSpec((DFF, H), lambda b: (0, 0)),
        pl.BlockSpec((DFF, H), lambda b: (0, 0)),
        pl.BlockSpec((H, DFF), lambda b: (0, 0)),
    ],
    out_specs=pl.BlockSpec((BT, H), lambda b: (b, 0)),
    out_shape=jax.ShapeDtypeStruct((T, H), F32),
)


def _piped_gather(src_hbm, chunks, bufs, sems, conds=None):
    nch = len(chunks)

    def guarded(c, fn):
        if conds is None:
            fn()
        else:
            pl.when(conds[c])(fn)

    def issue(c):
        idx_ref, _ = chunks[c]

        def start():
            pltpu.async_copy(src_hbm.at[idx_ref], bufs[c % 2], sems[c % 2])
            return None

        guarded(c, start)

    issue(0)
    for c in range(nch):
        if c + 1 < nch:
            issue(c + 1)

        def fin(c=c):
            pltpu.make_async_copy(src_hbm.at[pl.ds(0, 16)], bufs[c % 2],
                                  sems[c % 2]).wait()
            pltpu.sync_copy(bufs[c % 2], chunks[c][1])

        guarded(c, fin)


def _dispatch_body(pos_hbm, w_hbm, x_hbm, bexp_hbm, xg_out, ws_out,
                   pos_v, tok_v, w_v, ws_v, bexp_v, r0, r1, s0, s1):
    cid = lax.axis_index("c")
    sid = lax.axis_index("s")
    wid = sid * 2 + cid
    pltpu.sync_copy(pos_hbm, pos_v)
    pltpu.sync_copy(bexp_hbm, bexp_v)
    tail = bexp_v[pl.ds(NBPAD - 16, 16)]
    nact = jnp.max(jnp.where(lax.iota(I32, 16) == 15, tail, 0))
    nslot_act = nact * BT

    zero16 = jnp.zeros((16,), I32)

    def zero_body(i, _):
        tok_v[pl.ds(i * 16, 16)] = zero16
        return 0

    lax.fori_loop(0, NSLOT // 16, zero_body, 0)

    def scat_body(i, _):
        idx = pos_v[pl.ds(i * 16, 16)]
        j = lax.iota(I32, 16) + i * 16
        plsc.store_scatter(tok_v, [idx], j & (T - 1))
        return 0

    lax.fori_loop(0, (2 * T) // 16, scat_body, 0)

    @pl.when(wid == 0)
    def _():
        pltpu.sync_copy(w_hbm, w_v)

        def scw_body(i, _):
            idx = pos_v[pl.ds(i * 16, 16)]
            plsc.store_scatter(ws_v, [idx], w_v[pl.ds(i * 16, 16)])
            return 0

        lax.fori_loop(0, (2 * T) // 16, scw_body, 0)
        pltpu.sync_copy(ws_v, ws_out)

    base = wid * SLOTS_PER
    chunks = [(tok_v.at[pl.ds(base + c * 16, 16)],
               xg_out.at[pl.ds(base + c * 16, 16)])
              for c in range(SLOTS_PER // 16)]
    conds = [base + c * 16 < nslot_act for c in range(SLOTS_PER // 16)]
    _piped_gather(x_hbm, chunks, (r0, r1), (s0, s1), conds)


@functools.cache
def _get_dispatch():
    mesh = plsc.VectorSubcoreMesh(core_axis_name="c", subcore_axis_name="s")
    return pl.kernel(
        _dispatch_body,
        out_type=(
            jax.ShapeDtypeStruct((NSLOT, H // 2), F32),
            jax.ShapeDtypeStruct((NSLOT,), F32),
        ),
        mesh=mesh,
        compiler_params=pltpu.CompilerParams(needs_layout_passes=False),
        scratch_types=[
            pltpu.VMEM((2 * T,), I32),
            pltpu.VMEM((NSLOT,), I32),
            pltpu.VMEM((2 * T,), F32),
            pltpu.VMEM((NSLOT,), F32),
            pltpu.VMEM((NBPAD,), I32),
            pltpu.VMEM((16, H // 2), F32),
            pltpu.VMEM((16, H // 2), F32),
            pltpu.SemaphoreType.DMA,
            pltpu.SemaphoreType.DMA,
        ],
    )


def _combine_body(pos_hbm, yg_hbm, y01_out, p_v, r0, r1, s0, s1):
    cid = lax.axis_index("c")
    sid = lax.axis_index("s")
    wid = sid * 2 + cid
    t0 = wid * TOK_PER
    pltpu.sync_copy(pos_hbm.at[pl.ds(t0, TOK_PER)], p_v.at[pl.ds(0, TOK_PER)])
    pltpu.sync_copy(pos_hbm.at[pl.ds(T + t0, TOK_PER)],
                    p_v.at[pl.ds(TOK_PER, TOK_PER)])
    nch = TOK_PER // 16
    chunks = [(p_v.at[pl.ds(c * 16, 16)],
               y01_out.at[pl.ds(t0 + c * 16, 16)])
              for c in range(nch)]
    chunks += [(p_v.at[pl.ds(TOK_PER + c * 16, 16)],
                y01_out.at[pl.ds(T + t0 + c * 16, 16)])
               for c in range(nch)]
    _piped_gather(yg_hbm, chunks, (r0, r1), (s0, s1))


@functools.cache
def _get_combine():
    mesh = plsc.VectorSubcoreMesh(core_axis_name="c", subcore_axis_name="s")
    return pl.kernel(
        _combine_body,
        out_type=jax.ShapeDtypeStruct((2 * T, H), F32),
        mesh=mesh,
        compiler_params=pltpu.CompilerParams(needs_layout_passes=False),
        scratch_types=[
            pltpu.VMEM((2 * TOK_PER,), I32),
            pltpu.VMEM((16, H), F32),
            pltpu.VMEM((16, H), F32),
            pltpu.SemaphoreType.DMA,
            pltpu.SemaphoreType.DMA,
        ],
    )


BA = 256


def _add_body(sh_ref, y0_ref, y1_ref, out_ref):
    out_ref[...] = sh_ref[...] + y0_ref[...] + y1_ref[...]


_final_add = pl.pallas_call(
    _add_body,
    grid=(T // BA,),
    in_specs=[
        pl.BlockSpec((BA, H), lambda b: (b, 0)),
        pl.BlockSpec((BA, H), lambda b: (b, 0)),
        pl.BlockSpec((BA, H), lambda b: (b + T // BA, 0)),
    ],
    out_specs=pl.BlockSpec((BA, H), lambda b: (b, 0)),
    out_shape=jax.ShapeDtypeStruct((T, H), F32),
)


def kernel(hidden_states, gate_weight, gate_bias, gate_proj, up_proj,
           down_proj, shared_gate, shared_up, shared_down):
    Bv, Sv, Hv = hidden_states.shape
    x = hidden_states.reshape(T, H)
    pos2, w2, bexp, x16 = _router(x, gate_weight, gate_bias.reshape(E, 1))
    posf = pos2.reshape(2 * T)
    x16v = lax.bitcast_convert_type(x16.reshape(T, H // 2, 2), F32)
    xgv, wslot = _get_dispatch()(posf, w2.reshape(2 * T), x16v,
                                 bexp.reshape(NBPAD))
    xg16 = lax.bitcast_convert_type(xgv, BF16).reshape(NSLOT, H)
    yg = _ffn(bexp.reshape(NBPAD), xg16, gate_proj, up_proj, down_proj,
              wslot.reshape(NB, BT, 1))
    sh = _shared(x, shared_gate, shared_up, shared_down)
    y01 = _get_combine()(posf, yg)
    out = _final_add(sh, y01, y01)
    return out.reshape(Bv, Sv, Hv)


# speedup vs baseline: 1.8868x; 1.8868x over previous
import functools

import jax
import jax.numpy as jnp
from jax import lax
from jax.experimental import pallas as pl
from jax.experimental.pallas import tpu as pltpu
from jax.experimental.pallas import tpu_sc as plsc

E = 8
TOP_K = 2
N_GROUP = 4
TOPK_GROUP = 2
H = 2048
DFF = 1024
SCALE = 2.5
T = 2048
GSZ = E // N_GROUP

BT = 256
NB = 24
NSLOT = NB * BT
NBPAD = 64

NW = 32
SLOTS_PER = NSLOT // NW
TOK_PER = T // NW
F32 = jnp.float32
I32 = jnp.int32
BF16 = jnp.bfloat16


def _lane_cumsum(v):
    n = v.shape[1]
    sh = 1
    while sh < n:
        v = v + jnp.concatenate(
            [jnp.zeros((1, sh), v.dtype), v[:, : n - sh]], axis=1)
        sh *= 2
    return v


def _router_body(x_ref, gw_ref, gb_ref, pos_ref, w_ref, bexp_ref):
    x = x_ref[...]
    gw = gw_ref[...]
    logits = lax.dot_general(gw, x, (((1,), (1,)), ((), ())),
                             preferred_element_type=F32)
    scores = jax.nn.sigmoid(logits) + gb_ref[...]

    s = [scores[e:e + 1, :] for e in range(E)]
    g = [jnp.maximum(s[2 * k], s[2 * k + 1]) for k in range(N_GROUP)]

    gmask = []
    for a in range(N_GROUP):
        rank = jnp.zeros_like(g[a])
        for b in range(N_GROUP):
            if b == a:
                continue
            beats = (g[b] > g[a]) if b > a else (g[b] >= g[a])
            rank = rank + beats.astype(F32)
        gmask.append((rank < float(TOPK_GROUP)).astype(F32))

    ms = [s[e] * gmask[e // GSZ] for e in range(E)]

    erank = []
    for a in range(E):
        rank = jnp.zeros_like(ms[a])
        for b in range(E):
            if b == a:
                continue
            beats = (ms[b] > ms[a]) if b > a else (ms[b] >= ms[a])
            rank = rank + beats.astype(F32)
        erank.append(rank)

    m0 = [(erank[e] == 0.0).astype(F32) for e in range(E)]
    m1 = [(erank[e] == 1.0).astype(F32) for e in range(E)]
    w0 = sum(ms[e] * m0[e] for e in range(E))
    w1 = sum(ms[e] * m1[e] for e in range(E))
    denom = w0 + w1 + 1e-6
    w0n = w0 / denom * SCALE
    w1n = w1 / denom * SCALE

    c0ex, c1ex, tot0, cnt = [], [], [], []
    for e in range(E):
        i0 = _lane_cumsum(m0[e])
        i1 = _lane_cumsum(m1[e])
        c0ex.append(i0 - m0[e])
        c1ex.append(i1 - m1[e])
        tot0.append(i0[:, T - 1:T])
        cnt.append(i0[:, T - 1:T] + i1[:, T - 1:T])

    nblk, bstart = [], []
    run = jnp.zeros((1, 1), F32)
    for e in range(E):
        bstart.append(run)
        nblk.append(jnp.floor((cnt[e] + float(BT - 1)) / float(BT)))
        run = run + nblk[e]

    pos0 = sum(m0[e] * (bstart[e] * float(BT) + c0ex[e]) for e in range(E))
    pos1 = sum(m1[e] * (bstart[e] * float(BT) + tot0[e] + c1ex[e])
               for e in range(E))

    pos_ref[...] = jnp.concatenate([pos0, pos1], axis=0).astype(I32)
    w_ref[...] = jnp.concatenate([w0n, w1n], axis=0)

    biota = lax.broadcasted_iota(I32, (1, NBPAD), 1).astype(F32)
    bexp = jnp.zeros((1, NBPAD), F32)
    for e in range(E):
        inrange = (biota >= bstart[e]) & (biota < bstart[e] + nblk[e])
        bexp = bexp + float(e) * inrange.astype(F32)
    bexp = bexp + float(E - 1) * (biota >= run).astype(F32)
    bexp = jnp.where(biota == float(NBPAD - 1), run, bexp)
    bexp_ref[...] = bexp.astype(I32)


_router = pl.pallas_call(
    _router_body,
    out_shape=(
        jax.ShapeDtypeStruct((2, T), I32),
        jax.ShapeDtypeStruct((2, T), F32),
        jax.ShapeDtypeStruct((1, NBPAD), I32),
    ),
)


def _ffn_body(bexp_ref, xg_ref, gp_ref, up_ref, dp_ref, ws_ref, yg_ref):
    b = pl.program_id(0)

    @pl.when(b < bexp_ref[NBPAD - 1])
    def _():
        xb = xg_ref[...].astype(BF16)
        gcon = (((1,), (1,)), ((), ()))
        g = lax.dot_general(xb, gp_ref[0].astype(BF16), gcon,
                            preferred_element_type=F32)
        u = lax.dot_general(xb, up_ref[0].astype(BF16), gcon,
                            preferred_element_type=F32)
        h = (g * jax.nn.sigmoid(g) * u).astype(BF16)
        y = lax.dot_general(h, dp_ref[0].astype(BF16), gcon,
                            preferred_element_type=F32)
        yg_ref[...] = y * ws_ref[0]


_ffn = pl.pallas_call(
    _ffn_body,
    grid_spec=pltpu.PrefetchScalarGridSpec(
        num_scalar_prefetch=1,
        grid=(NB,),
        in_specs=[
            pl.BlockSpec((BT, H), lambda b, bexp: (b, 0)),
            pl.BlockSpec((1, DFF, H), lambda b, bexp: (bexp[b], 0, 0)),
            pl.BlockSpec((1, DFF, H), lambda b, bexp: (bexp[b], 0, 0)),
            pl.BlockSpec((1, H, DFF), lambda b, bexp: (bexp[b], 0, 0)),
            pl.BlockSpec((1, BT, 1), lambda b, bexp: (b, 0, 0)),
        ],
        out_specs=pl.BlockSpec((BT, H), lambda b, bexp: (b, 0)),
    ),
    out_shape=jax.ShapeDtypeStruct((NSLOT, H), F32),
)


def _shared_body(x_ref, sg_ref, su_ref, sd_ref, out_ref):
    xb = x_ref[...].astype(BF16)
    gcon = (((1,), (1,)), ((), ()))
    g = lax.dot_general(xb, sg_ref[...].astype(BF16), gcon,
                        preferred_element_type=F32)
    u = lax.dot_general(xb, su_ref[...].astype(BF16), gcon,
                        preferred_element_type=F32)
    h = (g * jax.nn.sigmoid(g) * u).astype(BF16)
    out_ref[...] = lax.dot_general(h, sd_ref[...].astype(BF16), gcon,
                                   preferred_element_type=F32)


_shared = pl.pallas_call(
    _shared_body,
    grid=(T // BT,),
    in_specs=[
        pl.BlockSpec((BT, H), lambda b: (b, 0)),
        pl.BlockSpec((DFF, H), lambda b: (0, 0)),
        pl.BlockSpec((DFF, H), lambda b: (0, 0)),
        pl.BlockSpec((H, DFF), lambda b: (0, 0)),
    ],
    out_specs=pl.BlockSpec((BT, H), lambda b: (b, 0)),
    out_shape=jax.ShapeDtypeStruct((T, H), F32),
)


def _piped_gather(src_hbm, chunks, bufs, sems, conds=None):
    nch = len(chunks)

    def guarded(c, fn):
        if conds is None:
            fn()
        else:
            pl.when(conds[c])(fn)

    def issue(c):
        idx_ref, _ = chunks[c]

        def start():
            pltpu.async_copy(src_hbm.at[idx_ref], bufs[c % 2], sems[c % 2])
            return None

        guarded(c, start)

    issue(0)
    for c in range(nch):
        if c + 1 < nch:
            issue(c + 1)

        def fin(c=c):
            pltpu.make_async_copy(src_hbm.at[pl.ds(0, 16)], bufs[c % 2],
                                  sems[c % 2]).wait()
            pltpu.sync_copy(bufs[c % 2], chunks[c][1])

        guarded(c, fin)


def _dispatch_body(pos_hbm, w_hbm, x_hbm, bexp_hbm, xg_out, ws_out,
                   pos_v, tok_v, w_v, ws_v, bexp_v, r0, r1, s0, s1):
    cid = lax.axis_index("c")
    sid = lax.axis_index("s")
    wid = sid * 2 + cid
    pltpu.sync_copy(pos_hbm, pos_v)
    pltpu.sync_copy(bexp_hbm, bexp_v)
    tail = bexp_v[pl.ds(NBPAD - 16, 16)]
    nact = jnp.max(jnp.where(lax.iota(I32, 16) == 15, tail, 0))
    nslot_act = nact * BT

    zero16 = jnp.zeros((16,), I32)

    def zero_body(i, _):
        tok_v[pl.ds(i * 16, 16)] = zero16
        return 0

    lax.fori_loop(0, NSLOT // 16, zero_body, 0)

    def scat_body(i, _):
        idx = pos_v[pl.ds(i * 16, 16)]
        j = lax.iota(I32, 16) + i * 16
        plsc.store_scatter(tok_v, [idx], j & (T - 1))
        return 0

    lax.fori_loop(0, (2 * T) // 16, scat_body, 0)

    @pl.when(wid == 0)
    def _():
        pltpu.sync_copy(w_hbm, w_v)

        def scw_body(i, _):
            idx = pos_v[pl.ds(i * 16, 16)]
            plsc.store_scatter(ws_v, [idx], w_v[pl.ds(i * 16, 16)])
            return 0

        lax.fori_loop(0, (2 * T) // 16, scw_body, 0)
        pltpu.sync_copy(ws_v, ws_out)

    base = wid * SLOTS_PER
    chunks = [(tok_v.at[pl.ds(base + c * 16, 16)],
               xg_out.at[pl.ds(base + c * 16, 16)])
              for c in range(SLOTS_PER // 16)]
    conds = [base + c * 16 < nslot_act for c in range(SLOTS_PER // 16)]
    _piped_gather(x_hbm, chunks, (r0, r1), (s0, s1), conds)


@functools.cache
def _get_dispatch():
    mesh = plsc.VectorSubcoreMesh(core_axis_name="c", subcore_axis_name="s")
    return pl.kernel(
        _dispatch_body,
        out_type=(
            jax.ShapeDtypeStruct((NSLOT, H), F32),
            jax.ShapeDtypeStruct((NSLOT,), F32),
        ),
        mesh=mesh,
        compiler_params=pltpu.CompilerParams(needs_layout_passes=False),
        scratch_types=[
            pltpu.VMEM((2 * T,), I32),
            pltpu.VMEM((NSLOT,), I32),
            pltpu.VMEM((2 * T,), F32),
            pltpu.VMEM((NSLOT,), F32),
            pltpu.VMEM((NBPAD,), I32),
            pltpu.VMEM((16, H), F32),
            pltpu.VMEM((16, H), F32),
            pltpu.SemaphoreType.DMA,
            pltpu.SemaphoreType.DMA,
        ],
    )


def _combine_body(pos_hbm, yg_hbm, y01_out, p_v, r0, r1, s0, s1):
    cid = lax.axis_index("c")
    sid = lax.axis_index("s")
    wid = sid * 2 + cid
    t0 = wid * TOK_PER
    pltpu.sync_copy(pos_hbm.at[pl.ds(t0, TOK_PER)], p_v.at[pl.ds(0, TOK_PER)])
    pltpu.sync_copy(pos_hbm.at[pl.ds(T + t0, TOK_PER)],
                    p_v.at[pl.ds(TOK_PER, TOK_PER)])
    nch = TOK_PER // 16
    chunks = [(p_v.at[pl.ds(c * 16, 16)],
               y01_out.at[pl.ds(t0 + c * 16, 16)])
              for c in range(nch)]
    chunks += [(p_v.at[pl.ds(TOK_PER + c * 16, 16)],
                y01_out.at[pl.ds(T + t0 + c * 16, 16)])
               for c in range(nch)]
    _piped_gather(yg_hbm, chunks, (r0, r1), (s0, s1))


@functools.cache
def _get_combine():
    mesh = plsc.VectorSubcoreMesh(core_axis_name="c", subcore_axis_name="s")
    return pl.kernel(
        _combine_body,
        out_type=jax.ShapeDtypeStruct((2 * T, H), F32),
        mesh=mesh,
        compiler_params=pltpu.CompilerParams(needs_layout_passes=False),
        scratch_types=[
            pltpu.VMEM((2 * TOK_PER,), I32),
            pltpu.VMEM((16, H), F32),
            pltpu.VMEM((16, H), F32),
            pltpu.SemaphoreType.DMA,
            pltpu.SemaphoreType.DMA,
        ],
    )


BA = 256


def _add_body(sh_ref, y0_ref, y1_ref, out_ref):
    out_ref[...] = sh_ref[...] + y0_ref[...] + y1_ref[...]


_final_add = pl.pallas_call(
    _add_body,
    grid=(T // BA,),
    in_specs=[
        pl.BlockSpec((BA, H), lambda b: (b, 0)),
        pl.BlockSpec((BA, H), lambda b: (b, 0)),
        pl.BlockSpec((BA, H), lambda b: (b + T // BA, 0)),
    ],
    out_specs=pl.BlockSpec((BA, H), lambda b: (b, 0)),
    out_shape=jax.ShapeDtypeStruct((T, H), F32),
)


def kernel(hidden_states, gate_weight, gate_bias, gate_proj, up_proj,
           down_proj, shared_gate, shared_up, shared_down):
    Bv, Sv, Hv = hidden_states.shape
    x = hidden_states.reshape(T, H)
    pos2, w2, bexp = _router(x, gate_weight, gate_bias.reshape(E, 1))
    posf = pos2.reshape(2 * T)
    xg, wslot = _get_dispatch()(posf, w2.reshape(2 * T), x,
                                bexp.reshape(NBPAD))
    yg = _ffn(bexp.reshape(NBPAD), xg, gate_proj, up_proj, down_proj,
              wslot.reshape(NB, BT, 1))
    sh = _shared(x, shared_gate, shared_up, shared_down)
    y01 = _get_combine()(posf, yg)
    out = _final_add(sh, y01, y01)
    return out.reshape(Bv, Sv, Hv)


# speedup vs baseline: 2.0323x; 1.0772x over previous
import functools

import jax
import jax.numpy as jnp
from jax import lax
from jax.experimental import pallas as pl
from jax.experimental.pallas import tpu as pltpu
from jax.experimental.pallas import tpu_sc as plsc

E = 8
TOP_K = 2
N_GROUP = 4
TOPK_GROUP = 2
H = 2048
DFF = 1024
SCALE = 2.5
T = 2048
GSZ = E // N_GROUP

BT = 256
NB = 24
NSLOT = NB * BT
NBPAD = 64

NW = 32
SLOTS_PER = NSLOT // NW
TOK_PER = T // NW
F32 = jnp.float32
I32 = jnp.int32
BF16 = jnp.bfloat16


def _lane_cumsum(v):
    n = v.shape[1]
    sh = 1
    while sh < n:
        v = v + jnp.concatenate(
            [jnp.zeros((1, sh), v.dtype), v[:, : n - sh]], axis=1)
        sh *= 2
    return v


def _router_body(x_ref, gw_ref, gb_ref, pos_ref, w_ref, bexp_ref, xp_ref):
    x = x_ref[...]
    gw = gw_ref[...]
    logits = lax.dot_general(gw, x, (((1,), (1,)), ((), ())),
                             preferred_element_type=F32)
    scores = jax.nn.sigmoid(logits) + gb_ref[...]

    s = [scores[e:e + 1, :] for e in range(E)]
    g = [jnp.maximum(s[2 * k], s[2 * k + 1]) for k in range(N_GROUP)]

    gmask = []
    for a in range(N_GROUP):
        rank = jnp.zeros_like(g[a])
        for b in range(N_GROUP):
            if b == a:
                continue
            beats = (g[b] > g[a]) if b > a else (g[b] >= g[a])
            rank = rank + beats.astype(F32)
        gmask.append((rank < float(TOPK_GROUP)).astype(F32))

    ms = [s[e] * gmask[e // GSZ] for e in range(E)]

    erank = []
    for a in range(E):
        rank = jnp.zeros_like(ms[a])
        for b in range(E):
            if b == a:
                continue
            beats = (ms[b] > ms[a]) if b > a else (ms[b] >= ms[a])
            rank = rank + beats.astype(F32)
        erank.append(rank)

    m0 = [(erank[e] == 0.0).astype(F32) for e in range(E)]
    m1 = [(erank[e] == 1.0).astype(F32) for e in range(E)]
    w0 = sum(ms[e] * m0[e] for e in range(E))
    w1 = sum(ms[e] * m1[e] for e in range(E))
    denom = w0 + w1 + 1e-6
    w0n = w0 / denom * SCALE
    w1n = w1 / denom * SCALE

    c0ex, c1ex, tot0, cnt = [], [], [], []
    for e in range(E):
        i0 = _lane_cumsum(m0[e])
        i1 = _lane_cumsum(m1[e])
        c0ex.append(i0 - m0[e])
        c1ex.append(i1 - m1[e])
        tot0.append(i0[:, T - 1:T])
        cnt.append(i0[:, T - 1:T] + i1[:, T - 1:T])

    nblk, bstart = [], []
    run = jnp.zeros((1, 1), F32)
    for e in range(E):
        bstart.append(run)
        nblk.append(jnp.floor((cnt[e] + float(BT - 1)) / float(BT)))
        run = run + nblk[e]

    pos0 = sum(m0[e] * (bstart[e] * float(BT) + c0ex[e]) for e in range(E))
    pos1 = sum(m1[e] * (bstart[e] * float(BT) + tot0[e] + c1ex[e])
               for e in range(E))

    pos_ref[...] = jnp.concatenate([pos0, pos1], axis=0).astype(I32)
    w_ref[...] = jnp.concatenate([w0n, w1n], axis=0)

    biota = lax.broadcasted_iota(I32, (1, NBPAD), 1).astype(F32)
    bexp = jnp.zeros((1, NBPAD), F32)
    for e in range(E):
        inrange = (biota >= bstart[e]) & (biota < bstart[e] + nblk[e])
        bexp = bexp + float(e) * inrange.astype(F32)
    bexp = bexp + float(E - 1) * (biota >= run).astype(F32)
    bexp = jnp.where(biota == float(NBPAD - 1), run, bexp)
    bexp_ref[...] = bexp.astype(I32)

    def b16(v):
        return lax.convert_element_type(
            lax.bitcast_convert_type(v.astype(BF16), jnp.int16),
            I32) & jnp.int32(0xFFFF)

    lo = b16(x[:, :H // 2])
    hi = b16(x[:, H // 2:])
    xp_ref[...] = lo | (hi << 16)


_router = pl.pallas_call(
    _router_body,
    out_shape=(
        jax.ShapeDtypeStruct((2, T), I32),
        jax.ShapeDtypeStruct((2, T), F32),
        jax.ShapeDtypeStruct((1, NBPAD), I32),
        jax.ShapeDtypeStruct((T, H // 2), I32),
    ),
)


def _ffn_body(bexp_ref, xg_ref, gp_ref, up_ref, dp_ref, ws_ref, yg_ref):
    b = pl.program_id(0)

    @pl.when(b < bexp_ref[NBPAD - 1])
    def _():
        p = xg_ref[...]
        lo = lax.bitcast_convert_type(p << 16, F32)
        hi = lax.bitcast_convert_type(p & jnp.int32(-65536), F32)
        xb = jnp.concatenate([lo, hi], axis=1).astype(BF16)
        gcon = (((1,), (1,)), ((), ()))
        g = lax.dot_general(xb, gp_ref[0].astype(BF16), gcon,
                            preferred_element_type=F32)
        u = lax.dot_general(xb, up_ref[0].astype(BF16), gcon,
                            preferred_element_type=F32)
        h = (g * jax.nn.sigmoid(g) * u).astype(BF16)
        y = lax.dot_general(h, dp_ref[0].astype(BF16), gcon,
                            preferred_element_type=F32)
        yg_ref[...] = y * ws_ref[0]


_ffn = pl.pallas_call(
    _ffn_body,
    grid_spec=pltpu.PrefetchScalarGridSpec(
        num_scalar_prefetch=1,
        grid=(NB,),
        in_specs=[
            pl.BlockSpec((BT, H // 2), lambda b, bexp: (b, 0)),
            pl.BlockSpec((1, DFF, H), lambda b, bexp: (bexp[b], 0, 0)),
            pl.BlockSpec((1, DFF, H), lambda b, bexp: (bexp[b], 0, 0)),
            pl.BlockSpec((1, H, DFF), lambda b, bexp: (bexp[b], 0, 0)),
            pl.BlockSpec((1, BT, 1), lambda b, bexp: (b, 0, 0)),
        ],
        out_specs=pl.BlockSpec((BT, H), lambda b, bexp: (b, 0)),
    ),
    out_shape=jax.ShapeDtypeStruct((NSLOT, H), F32),
)


def _shared_body(x_ref, sg_ref, su_ref, sd_ref, out_ref):
    xb = x_ref[...].astype(BF16)
    gcon = (((1,), (1,)), ((), ()))
    g = lax.dot_general(xb, sg_ref[...].astype(BF16), gcon,
                        preferred_element_type=F32)
    u = lax.dot_general(xb, su_ref[...].astype(BF16), gcon,
                        preferred_element_type=F32)
    h = (g * jax.nn.sigmoid(g) * u).astype(BF16)
    out_ref[...] = lax.dot_general(h, sd_ref[...].astype(BF16), gcon,
                                   preferred_element_type=F32)


_shared = pl.pallas_call(
    _shared_body,
    grid=(T // BT,),
    in_specs=[
        pl.BlockSpec((BT, H), lambda b: (b, 0)),
        pl.BlockSpec((DFF, H), lambda b: (0, 0)),
        pl.BlockSpec((DFF, H), lambda b: (0, 0)),
        pl.BlockSpec((H, DFF), lambda b: (0, 0)),
    ],
    out_specs=pl.BlockSpec((BT, H), lambda b: (b, 0)),
    out_shape=jax.ShapeDtypeStruct((T, H), F32),
)


def _piped_gather(src_hbm, chunks, bufs, sems, conds=None):
    nch = len(chunks)

    def guarded(c, fn):
        if conds is None:
            fn()
        else:
            pl.when(conds[c])(fn)

    def issue(c):
        idx_ref, _ = chunks[c]

        def start():
            pltpu.async_copy(src_hbm.at[idx_ref], bufs[c % 2], sems[c % 2])
            return None

        guarded(c, start)

    issue(0)
    for c in range(nch):
        if c + 1 < nch:
            issue(c + 1)

        def fin(c=c):
            pltpu.make_async_copy(src_hbm.at[pl.ds(0, 16)], bufs[c % 2],
                                  sems[c % 2]).wait()
            pltpu.sync_copy(bufs[c % 2], chunks[c][1])

        guarded(c, fin)


def _dispatch_body(pos_hbm, w_hbm, x_hbm, bexp_hbm, xg_out, ws_out,
                   pos_v, tok_v, w_v, ws_v, bexp_v, r0, r1, s0, s1):
    cid = lax.axis_index("c")
    sid = lax.axis_index("s")
    wid = sid * 2 + cid
    pltpu.sync_copy(pos_hbm, pos_v)
    pltpu.sync_copy(bexp_hbm, bexp_v)
    tail = bexp_v[pl.ds(NBPAD - 16, 16)]
    nact = jnp.max(jnp.where(lax.iota(I32, 16) == 15, tail, 0))
    nslot_act = nact * BT

    zero16 = jnp.zeros((16,), I32)

    def zero_body(i, _):
        tok_v[pl.ds(i * 16, 16)] = zero16
        return 0

    lax.fori_loop(0, NSLOT // 16, zero_body, 0)

    def scat_body(i, _):
        idx = pos_v[pl.ds(i * 16, 16)]
        j = lax.iota(I32, 16) + i * 16
        plsc.store_scatter(tok_v, [idx], j & (T - 1))
        return 0

    lax.fori_loop(0, (2 * T) // 16, scat_body, 0)

    @pl.when(wid == 0)
    def _():
        pltpu.sync_copy(w_hbm, w_v)

        def scw_body(i, _):
            idx = pos_v[pl.ds(i * 16, 16)]
            plsc.store_scatter(ws_v, [idx], w_v[pl.ds(i * 16, 16)])
            return 0

        lax.fori_loop(0, (2 * T) // 16, scw_body, 0)
        pltpu.sync_copy(ws_v, ws_out)

    base = wid * SLOTS_PER
    chunks = [(tok_v.at[pl.ds(base + c * 16, 16)],
               xg_out.at[pl.ds(base + c * 16, 16)])
              for c in range(SLOTS_PER // 16)]
    conds = [base + c * 16 < nslot_act for c in range(SLOTS_PER // 16)]
    _piped_gather(x_hbm, chunks, (r0, r1), (s0, s1), conds)


@functools.cache
def _get_dispatch():
    mesh = plsc.VectorSubcoreMesh(core_axis_name="c", subcore_axis_name="s")
    return pl.kernel(
        _dispatch_body,
        out_type=(
            jax.ShapeDtypeStruct((NSLOT, H // 2), I32),
            jax.ShapeDtypeStruct((NSLOT,), F32),
        ),
        mesh=mesh,
        compiler_params=pltpu.CompilerParams(needs_layout_passes=False),
        scratch_types=[
            pltpu.VMEM((2 * T,), I32),
            pltpu.VMEM((NSLOT,), I32),
            pltpu.VMEM((2 * T,), F32),
            pltpu.VMEM((NSLOT,), F32),
            pltpu.VMEM((NBPAD,), I32),
            pltpu.VMEM((16, H // 2), I32),
            pltpu.VMEM((16, H // 2), I32),
            pltpu.SemaphoreType.DMA,
            pltpu.SemaphoreType.DMA,
        ],
    )


def _combine_body(pos_hbm, yg_hbm, y01_out, p_v, r0, r1, s0, s1):
    cid = lax.axis_index("c")
    sid = lax.axis_index("s")
    wid = sid * 2 + cid
    t0 = wid * TOK_PER
    pltpu.sync_copy(pos_hbm.at[pl.ds(t0, TOK_PER)], p_v.at[pl.ds(0, TOK_PER)])
    pltpu.sync_copy(pos_hbm.at[pl.ds(T + t0, TOK_PER)],
                    p_v.at[pl.ds(TOK_PER, TOK_PER)])
    nch = TOK_PER // 16
    chunks = [(p_v.at[pl.ds(c * 16, 16)],
               y01_out.at[pl.ds(t0 + c * 16, 16)])
              for c in range(nch)]
    chunks += [(p_v.at[pl.ds(TOK_PER + c * 16, 16)],
                y01_out.at[pl.ds(T + t0 + c * 16, 16)])
               for c in range(nch)]
    _piped_gather(yg_hbm, chunks, (r0, r1), (s0, s1))


@functools.cache
def _get_combine():
    mesh = plsc.VectorSubcoreMesh(core_axis_name="c", subcore_axis_name="s")
    return pl.kernel(
        _combine_body,
        out_type=jax.ShapeDtypeStruct((2 * T, H), F32),
        mesh=mesh,
        compiler_params=pltpu.CompilerParams(needs_layout_passes=False),
        scratch_types=[
            pltpu.VMEM((2 * TOK_PER,), I32),
            pltpu.VMEM((16, H), F32),
            pltpu.VMEM((16, H), F32),
            pltpu.SemaphoreType.DMA,
            pltpu.SemaphoreType.DMA,
        ],
    )


BA = 256


def _add_body(sh_ref, y0_ref, y1_ref, out_ref):
    out_ref[...] = sh_ref[...] + y0_ref[...] + y1_ref[...]


_final_add = pl.pallas_call(
    _add_body,
    grid=(T // BA,),
    in_specs=[
        pl.BlockSpec((BA, H), lambda b: (b, 0)),
        pl.BlockSpec((BA, H), lambda b: (b, 0)),
        pl.BlockSpec((BA, H), lambda b: (b + T // BA, 0)),
    ],
    out_specs=pl.BlockSpec((BA, H), lambda b: (b, 0)),
    out_shape=jax.ShapeDtypeStruct((T, H), F32),
)


def kernel(hidden_states, gate_weight, gate_bias, gate_proj, up_proj,
           down_proj, shared_gate, shared_up, shared_down):
    Bv, Sv, Hv = hidden_states.shape
    x = hidden_states.reshape(T, H)
    pos2, w2, bexp, xp = _router(x, gate_weight, gate_bias.reshape(E, 1))
    posf = pos2.reshape(2 * T)
    xg, wslot = _get_dispatch()(posf, w2.reshape(2 * T), xp,
                                bexp.reshape(NBPAD))
    yg = _ffn(bexp.reshape(NBPAD), xg, gate_proj, up_proj, down_proj,
              wslot.reshape(NB, BT, 1))
    sh = _shared(x, shared_gate, shared_up, shared_down)
    y01 = _get_combine()(posf, yg)
    out = _final_add(sh, y01, y01)
    return out.reshape(Bv, Sv, Hv)


# speedup vs baseline: 2.0829x; 1.0249x over previous
import functools

import jax
import jax.numpy as jnp
from jax import lax
from jax.experimental import pallas as pl
from jax.experimental.pallas import tpu as pltpu
from jax.experimental.pallas import tpu_sc as plsc

E = 8
TOP_K = 2
N_GROUP = 4
TOPK_GROUP = 2
H = 2048
DFF = 1024
SCALE = 2.5
T = 2048
GSZ = E // N_GROUP

BT = 256
NB = 24
NSLOT = NB * BT
NBPAD = 64

NW = 32
SLOTS_PER = NSLOT // NW
TOK_PER = T // NW
F32 = jnp.float32
I32 = jnp.int32
BF16 = jnp.bfloat16


def _pack16(v):
    def b16(u):
        return lax.convert_element_type(
            lax.bitcast_convert_type(u.astype(BF16), jnp.int16),
            I32) & jnp.int32(0xFFFF)

    half = v.shape[1] // 2
    return b16(v[:, :half]) | (b16(v[:, half:]) << 16)


def _unpack16(p):
    lo = lax.bitcast_convert_type(p << 16, F32)
    hi = lax.bitcast_convert_type(p & jnp.int32(-65536), F32)
    return jnp.concatenate([lo, hi], axis=1)


def _lane_cumsum(v):
    n = v.shape[1]
    sh = 1
    while sh < n:
        v = v + jnp.concatenate(
            [jnp.zeros((1, sh), v.dtype), v[:, : n - sh]], axis=1)
        sh *= 2
    return v


def _router_body(x_ref, gw_ref, gb_ref, pos_ref, w_ref, bexp_ref, xp_ref):
    x = x_ref[...]
    gw = gw_ref[...]
    logits = lax.dot_general(gw, x, (((1,), (1,)), ((), ())),
                             preferred_element_type=F32)
    scores = jax.nn.sigmoid(logits) + gb_ref[...]

    s = [scores[e:e + 1, :] for e in range(E)]
    g = [jnp.maximum(s[2 * k], s[2 * k + 1]) for k in range(N_GROUP)]

    gmask = []
    for a in range(N_GROUP):
        rank = jnp.zeros_like(g[a])
        for b in range(N_GROUP):
            if b == a:
                continue
            beats = (g[b] > g[a]) if b > a else (g[b] >= g[a])
            rank = rank + beats.astype(F32)
        gmask.append((rank < float(TOPK_GROUP)).astype(F32))

    ms = [s[e] * gmask[e // GSZ] for e in range(E)]

    erank = []
    for a in range(E):
        rank = jnp.zeros_like(ms[a])
        for b in range(E):
            if b == a:
                continue
            beats = (ms[b] > ms[a]) if b > a else (ms[b] >= ms[a])
            rank = rank + beats.astype(F32)
        erank.append(rank)

    m0 = [(erank[e] == 0.0).astype(F32) for e in range(E)]
    m1 = [(erank[e] == 1.0).astype(F32) for e in range(E)]
    w0 = sum(ms[e] * m0[e] for e in range(E))
    w1 = sum(ms[e] * m1[e] for e in range(E))
    denom = w0 + w1 + 1e-6
    w0n = w0 / denom * SCALE
    w1n = w1 / denom * SCALE

    c0ex, c1ex, tot0, cnt = [], [], [], []
    for e in range(E):
        i0 = _lane_cumsum(m0[e])
        i1 = _lane_cumsum(m1[e])
        c0ex.append(i0 - m0[e])
        c1ex.append(i1 - m1[e])
        tot0.append(i0[:, T - 1:T])
        cnt.append(i0[:, T - 1:T] + i1[:, T - 1:T])

    nblk, bstart = [], []
    run = jnp.zeros((1, 1), F32)
    for e in range(E):
        bstart.append(run)
        nblk.append(jnp.floor((cnt[e] + float(BT - 1)) / float(BT)))
        run = run + nblk[e]

    pos0 = sum(m0[e] * (bstart[e] * float(BT) + c0ex[e]) for e in range(E))
    pos1 = sum(m1[e] * (bstart[e] * float(BT) + tot0[e] + c1ex[e])
               for e in range(E))

    pos_ref[...] = jnp.concatenate([pos0, pos1], axis=0).astype(I32)
    w_ref[...] = jnp.concatenate([w0n, w1n], axis=0)

    biota = lax.broadcasted_iota(I32, (1, NBPAD), 1).astype(F32)
    bexp = jnp.zeros((1, NBPAD), F32)
    for e in range(E):
        inrange = (biota >= bstart[e]) & (biota < bstart[e] + nblk[e])
        bexp = bexp + float(e) * inrange.astype(F32)
    bexp = bexp + float(E - 1) * (biota >= run).astype(F32)
    bexp = jnp.where(biota == float(NBPAD - 1), run, bexp)
    bexp_ref[...] = bexp.astype(I32)

    xp_ref[...] = _pack16(x)


_router = pl.pallas_call(
    _router_body,
    out_shape=(
        jax.ShapeDtypeStruct((2, T), I32),
        jax.ShapeDtypeStruct((2, T), F32),
        jax.ShapeDtypeStruct((1, NBPAD), I32),
        jax.ShapeDtypeStruct((T, H // 2), I32),
    ),
)


def _ffn_body(bexp_ref, xg_ref, gp_ref, up_ref, dp_ref, ws_ref, yg_ref):
    b = pl.program_id(0)

    @pl.when(b < bexp_ref[NBPAD - 1])
    def _():
        xb = _unpack16(xg_ref[...]).astype(BF16)
        gcon = (((1,), (1,)), ((), ()))
        g = lax.dot_general(xb, gp_ref[0].astype(BF16), gcon,
                            preferred_element_type=F32)
        u = lax.dot_general(xb, up_ref[0].astype(BF16), gcon,
                            preferred_element_type=F32)
        h = (g * jax.nn.sigmoid(g) * u).astype(BF16)
        y = lax.dot_general(h, dp_ref[0].astype(BF16), gcon,
                            preferred_element_type=F32)
        yg_ref[...] = _pack16(y * ws_ref[0])


_ffn = pl.pallas_call(
    _ffn_body,
    grid_spec=pltpu.PrefetchScalarGridSpec(
        num_scalar_prefetch=1,
        grid=(NB,),
        in_specs=[
            pl.BlockSpec((BT, H // 2), lambda b, bexp: (b, 0)),
            pl.BlockSpec((1, DFF, H), lambda b, bexp: (bexp[b], 0, 0)),
            pl.BlockSpec((1, DFF, H), lambda b, bexp: (bexp[b], 0, 0)),
            pl.BlockSpec((1, H, DFF), lambda b, bexp: (bexp[b], 0, 0)),
            pl.BlockSpec((1, BT, 1), lambda b, bexp: (b, 0, 0)),
        ],
        out_specs=pl.BlockSpec((BT, H // 2), lambda b, bexp: (b, 0)),
    ),
    out_shape=jax.ShapeDtypeStruct((NSLOT, H // 2), I32),
)


def _shared_body(x_ref, sg_ref, su_ref, sd_ref, out_ref):
    xb = x_ref[...].astype(BF16)
    gcon = (((1,), (1,)), ((), ()))
    g = lax.dot_general(xb, sg_ref[...].astype(BF16), gcon,
                        preferred_element_type=F32)
    u = lax.dot_general(xb, su_ref[...].astype(BF16), gcon,
                        preferred_element_type=F32)
    h = (g * jax.nn.sigmoid(g) * u).astype(BF16)
    out_ref[...] = lax.dot_general(h, sd_ref[...].astype(BF16), gcon,
                                   preferred_element_type=F32)


_shared = pl.pallas_call(
    _shared_body,
    grid=(T // BT,),
    in_specs=[
        pl.BlockSpec((BT, H), lambda b: (b, 0)),
        pl.BlockSpec((DFF, H), lambda b: (0, 0)),
        pl.BlockSpec((DFF, H), lambda b: (0, 0)),
        pl.BlockSpec((H, DFF), lambda b: (0, 0)),
    ],
    out_specs=pl.BlockSpec((BT, H), lambda b: (b, 0)),
    out_shape=jax.ShapeDtypeStruct((T, H), F32),
)


def _piped_gather(src_hbm, chunks, bufs, sems, conds=None):
    nch = len(chunks)

    def guarded(c, fn):
        if conds is None:
            fn()
        else:
            pl.when(conds[c])(fn)

    def issue(c):
        idx_ref, _ = chunks[c]

        def start():
            pltpu.async_copy(src_hbm.at[idx_ref], bufs[c % 2], sems[c % 2])
            return None

        guarded(c, start)

    issue(0)
    for c in range(nch):
        if c + 1 < nch:
            issue(c + 1)

        def fin(c=c):
            pltpu.make_async_copy(src_hbm.at[pl.ds(0, 16)], bufs[c % 2],
                                  sems[c % 2]).wait()
            pltpu.sync_copy(bufs[c % 2], chunks[c][1])

        guarded(c, fin)


def _dispatch_body(pos_hbm, w_hbm, x_hbm, bexp_hbm, xg_out, ws_out,
                   pos_v, tok_v, w_v, ws_v, bexp_v, r0, r1, s0, s1):
    cid = lax.axis_index("c")
    sid = lax.axis_index("s")
    wid = sid * 2 + cid
    pltpu.sync_copy(pos_hbm, pos_v)
    pltpu.sync_copy(bexp_hbm, bexp_v)
    tail = bexp_v[pl.ds(NBPAD - 16, 16)]
    nact = jnp.max(jnp.where(lax.iota(I32, 16) == 15, tail, 0))
    nslot_act = nact * BT

    zero16 = jnp.zeros((16,), I32)

    def zero_body(i, _):
        tok_v[pl.ds(i * 16, 16)] = zero16
        return 0

    lax.fori_loop(0, NSLOT // 16, zero_body, 0)

    def scat_body(i, _):
        idx = pos_v[pl.ds(i * 16, 16)]
        j = lax.iota(I32, 16) + i * 16
        plsc.store_scatter(tok_v, [idx], j & (T - 1))
        return 0

    lax.fori_loop(0, (2 * T) // 16, scat_body, 0)

    @pl.when(wid == 0)
    def _():
        pltpu.sync_copy(w_hbm, w_v)

        def scw_body(i, _):
            idx = pos_v[pl.ds(i * 16, 16)]
            plsc.store_scatter(ws_v, [idx], w_v[pl.ds(i * 16, 16)])
            return 0

        lax.fori_loop(0, (2 * T) // 16, scw_body, 0)
        pltpu.sync_copy(ws_v, ws_out)

    base = wid * SLOTS_PER
    chunks = [(tok_v.at[pl.ds(base + c * 16, 16)],
               xg_out.at[pl.ds(base + c * 16, 16)])
              for c in range(SLOTS_PER // 16)]
    conds = [base + c * 16 < nslot_act for c in range(SLOTS_PER // 16)]
    _piped_gather(x_hbm, chunks, (r0, r1), (s0, s1), conds)


@functools.cache
def _get_dispatch():
    mesh = plsc.VectorSubcoreMesh(core_axis_name="c", subcore_axis_name="s")
    return pl.kernel(
        _dispatch_body,
        out_type=(
            jax.ShapeDtypeStruct((NSLOT, H // 2), I32),
            jax.ShapeDtypeStruct((NSLOT,), F32),
        ),
        mesh=mesh,
        compiler_params=pltpu.CompilerParams(needs_layout_passes=False),
        scratch_types=[
            pltpu.VMEM((2 * T,), I32),
            pltpu.VMEM((NSLOT,), I32),
            pltpu.VMEM((2 * T,), F32),
            pltpu.VMEM((NSLOT,), F32),
            pltpu.VMEM((NBPAD,), I32),
            pltpu.VMEM((16, H // 2), I32),
            pltpu.VMEM((16, H // 2), I32),
            pltpu.SemaphoreType.DMA,
            pltpu.SemaphoreType.DMA,
        ],
    )


def _combine_body(pos_hbm, yg_hbm, y01_out, p_v, r0, r1, s0, s1):
    cid = lax.axis_index("c")
    sid = lax.axis_index("s")
    wid = sid * 2 + cid
    t0 = wid * TOK_PER
    pltpu.sync_copy(pos_hbm.at[pl.ds(t0, TOK_PER)], p_v.at[pl.ds(0, TOK_PER)])
    pltpu.sync_copy(pos_hbm.at[pl.ds(T + t0, TOK_PER)],
                    p_v.at[pl.ds(TOK_PER, TOK_PER)])
    nch = TOK_PER // 16
    chunks = [(p_v.at[pl.ds(c * 16, 16)],
               y01_out.at[pl.ds(t0 + c * 16, 16)])
              for c in range(nch)]
    chunks += [(p_v.at[pl.ds(TOK_PER + c * 16, 16)],
                y01_out.at[pl.ds(T + t0 + c * 16, 16)])
               for c in range(nch)]
    _piped_gather(yg_hbm, chunks, (r0, r1), (s0, s1))


@functools.cache
def _get_combine():
    mesh = plsc.VectorSubcoreMesh(core_axis_name="c", subcore_axis_name="s")
    return pl.kernel(
        _combine_body,
        out_type=jax.ShapeDtypeStruct((2 * T, H // 2), I32),
        mesh=mesh,
        compiler_params=pltpu.CompilerParams(needs_layout_passes=False),
        scratch_types=[
            pltpu.VMEM((2 * TOK_PER,), I32),
            pltpu.VMEM((16, H // 2), I32),
            pltpu.VMEM((16, H // 2), I32),
            pltpu.SemaphoreType.DMA,
            pltpu.SemaphoreType.DMA,
        ],
    )


BA = 256


def _add_body(sh_ref, y0_ref, y1_ref, out_ref):
    out_ref[...] = (sh_ref[...] + _unpack16(y0_ref[...])
                    + _unpack16(y1_ref[...]))


_final_add = pl.pallas_call(
    _add_body,
    grid=(T // BA,),
    in_specs=[
        pl.BlockSpec((BA, H), lambda b: (b, 0)),
        pl.BlockSpec((BA, H // 2), lambda b: (b, 0)),
        pl.BlockSpec((BA, H // 2), lambda b: (b + T // BA, 0)),
    ],
    out_specs=pl.BlockSpec((BA, H), lambda b: (b, 0)),
    out_shape=jax.ShapeDtypeStruct((T, H), F32),
)


def kernel(hidden_states, gate_weight, gate_bias, gate_proj, up_proj,
           down_proj, shared_gate, shared_up, shared_down):
    Bv, Sv, Hv = hidden_states.shape
    x = hidden_states.reshape(T, H)
    pos2, w2, bexp, xp = _router(x, gate_weight, gate_bias.reshape(E, 1))
    posf = pos2.reshape(2 * T)
    xg, wslot = _get_dispatch()(posf, w2.reshape(2 * T), xp,
                                bexp.reshape(NBPAD))
    yg = _ffn(bexp.reshape(NBPAD), xg, gate_proj, up_proj, down_proj,
              wslot.reshape(NB, BT, 1))
    sh = _shared(x, shared_gate, shared_up, shared_down)
    y01 = _get_combine()(posf, yg)
    out = _final_add(sh, y01, y01)
    return out.reshape(Bv, Sv, Hv)
